# baseline (speedup 1.0000x reference)
import functools

import jax
import jax.numpy as jnp
from jax import lax
from jax.experimental import pallas as pl
from jax.experimental.pallas import tpu as pltpu
from jax.experimental.pallas import tpu_sc as plsc

_NC = 2
_NS = 16
_NW = _NC * _NS
_L = 16

_B = 16384
_F = 26
_D = 64
_ND = _D // _L
_ROWS_PER_W = _B // _NW
_CHUNK = 8
_NCHUNK = _ROWS_PER_W // _CHUNK
_GROWS = _CHUNK * _F
_CBLK = 128
_NBLK = _ROWS_PER_W // _CBLK
_FLAT = _ROWS_PER_W * _F + _L // 2

_V = 1000000
_TC_C = 2048
_SPLIT = 501760
_TC_NBLK = _SPLIT // _TC_C
_TC_LASTBLK = (_V - 1) // _TC_C


def _detrans_body(a_ref, b_ref, out_ref):
    out_ref[:, 0:_D] = a_ref[...].T
    out_ref[:, _D:2 * _D] = b_ref[...].T


def _detranspose(table):
    tt = table.T
    paired = pl.pallas_call(
        _detrans_body,
        grid=(_TC_NBLK,),
        in_specs=[
            pl.BlockSpec((_D, _TC_C), lambda i: (0, i)),
            pl.BlockSpec((_D, _TC_C),
                         lambda i: (0, jnp.minimum(i + _TC_NBLK, _TC_LASTBLK))),
        ],
        out_specs=pl.BlockSpec((_TC_C, 2 * _D), lambda i: (i, 0)),
        out_shape=jax.ShapeDtypeStruct((_SPLIT, 2 * _D), jnp.float32),
    )(tt, tt)
    return paired.reshape(2 * _SPLIT, _D)


def _fm_body(uft, ift, ue, ie, w, out,
             stage_u, stage_i, flat_u, flat_i,
             u_rows, i_rows, w_v, part, scores, sem_a, sem_b, sem_o):
    wid = lax.axis_index("s") * _NC + lax.axis_index("c")
    base = wid * _ROWS_PER_W

    pltpu.sync_copy(w, w_v)
    wb = [w_v[pl.ds(k * _L, _L)] for k in range(_ND)]
    lane = lax.iota(jnp.int32, _L)
    lo_rows = lane
    hi_rows = jnp.minimum(lane + _L, _F - 1)

    for blk in range(_NBLK):
        col0 = base + blk * _CBLK
        pltpu.sync_copy(uft.at[:, pl.ds(col0, _CBLK)], stage_u)
        pltpu.sync_copy(ift.at[:, pl.ds(col0, _CBLK)], stage_i)

        def remap(r):
            return jnp.where(r < _SPLIT, r * 2, r * 2 - (2 * _SPLIT - 1))

        def reorder_body(b2, carry, blk=blk):
            col = jnp.full((_L,), b2, jnp.int32)
            b = blk * _CBLK + b2
            off = b * _F
            flat_u[pl.ds(off, _L)] = remap(
                plsc.load_gather(stage_u, [lo_rows, col]))
            flat_u[pl.ds(off + _L, _L)] = remap(
                plsc.load_gather(stage_u, [hi_rows, col]))
            flat_i[pl.ds(off, _L)] = remap(
                plsc.load_gather(stage_i, [lo_rows, col]))
            flat_i[pl.ds(off + _L, _L)] = remap(
                plsc.load_gather(stage_i, [hi_rows, col]))
            return carry

        lax.fori_loop(0, _CBLK, reorder_body, 0)

    sems = (sem_a, sem_b)

    def issue(c, p):
        off = c * _GROWS
        pltpu.async_copy(ue.at[flat_u.at[pl.ds(off, _GROWS)]], u_rows.at[p],
                         sems[p])
        pltpu.async_copy(ie.at[flat_i.at[pl.ds(off, _GROWS)]], i_rows.at[p],
                         sems[p])

    def drain(p):
        pltpu.make_async_copy(ue.at[pl.ds(0, _GROWS)], u_rows.at[p],
                              sems[p]).wait()
        pltpu.make_async_copy(ie.at[pl.ds(0, _GROWS)], i_rows.at[p],
                              sems[p]).wait()

    issue(0, 0)

    def super_body(s, carry):
        for par in range(2):
            c = s * 2 + par

            @pl.when(c < _NCHUNK - 1)
            def _():
                issue(c + 1, 1 - par)

            drain(par)

            def row_body(b, carry2, par=par):
                rb = b * _F
                ua = [jnp.zeros((_L,), jnp.float32) for _ in range(_ND)]
                sa = [jnp.zeros((_L,), jnp.float32) for _ in range(_ND)]
                qa = [jnp.zeros((_L,), jnp.float32) for _ in range(_ND)]
                for f in range(_F):
                    for k in range(_ND):
                        ua[k] = ua[k] + u_rows[par, rb + f, pl.ds(k * _L, _L)]
                for f in range(_F):
                    for k in range(_ND):
                        x = i_rows[par, rb + f, pl.ds(k * _L, _L)]
                        sa[k] = sa[k] + x
                        qa[k] = qa[k] + x * x
                tot = jnp.zeros((_L,), jnp.float32)
                for k in range(_ND):
                    tot = tot + (ua[k] + wb[k]) * sa[k] \
                        + 0.5 * (sa[k] * sa[k] - qa[k])
                col = jnp.full((_L,), par * _CHUNK + b, jnp.int32)
                plsc.store_scatter(part, [lane, col], tot)
                return carry2

            lax.fori_loop(0, _CHUNK, row_body, 0)

        svec = part[0, :]
        for l in range(1, _L):
            svec = svec + part[l, :]
        scores[pl.ds(s * _L, _L)] = svec
        return carry

    lax.fori_loop(0, _NCHUNK // 2, super_body, 0)
    pltpu.async_copy(scores, out.at[pl.ds(base, _ROWS_PER_W)], sem_o).wait()


@jax.jit
def kernel(user_features, item_features, user_emb, item_emb, linear_w):
    mesh = plsc.VectorSubcoreMesh(core_axis_name="c", subcore_axis_name="s",
                                  num_cores=_NC, num_subcores=_NS)
    fm = functools.partial(
        pl.kernel,
        out_type=jax.ShapeDtypeStruct((_B,), jnp.float32),
        mesh=mesh,
        compiler_params=pltpu.CompilerParams(needs_layout_passes=False,
                                             use_tc_tiling_on_sc=False),
        scratch_types=[
            pltpu.VMEM((_F, _CBLK), jnp.int32),
            pltpu.VMEM((_F, _CBLK), jnp.int32),
            pltpu.VMEM((_FLAT,), jnp.int32),
            pltpu.VMEM((_FLAT,), jnp.int32),
            pltpu.VMEM((2, _GROWS, _D), jnp.float32),
            pltpu.VMEM((2, _GROWS, _D), jnp.float32),
            pltpu.VMEM((_D,), jnp.float32),
            pltpu.VMEM((_L, 2 * _CHUNK), jnp.float32),
            pltpu.VMEM((_ROWS_PER_W,), jnp.float32),
            pltpu.SemaphoreType.DMA,
            pltpu.SemaphoreType.DMA,
            pltpu.SemaphoreType.DMA,
        ],
    )(_fm_body)
    return fm(user_features.T, item_features.T, _detranspose(user_emb),
              _detranspose(item_emb), linear_w.reshape(-1))


# speedup vs baseline: 1.3357x; 1.3357x over previous
import functools

import jax
import jax.numpy as jnp
from jax import lax
from jax.experimental import pallas as pl
from jax.experimental.pallas import tpu as pltpu
from jax.experimental.pallas import tpu_sc as plsc

_NC = 2
_NS = 16
_NW = _NC * _NS
_L = 16

_B = 16384
_F = 26
_D = 64
_ND = _D // _L
_ROWS_PER_W = _B // _NW
_CHUNK = 8
_NCHUNK = _ROWS_PER_W // _CHUNK
_GROWS = _CHUNK * _F
_CBLK = 128
_NBLK = _ROWS_PER_W // _CBLK
_FLAT = _ROWS_PER_W * _F + _L // 2

_V = 1000000
_TC_C = 8192
_SPLIT = 507904
_TC_NBLK = _SPLIT // _TC_C
_TC_LASTBLK = (_V - 1) // _TC_C


def _detrans_body(a_ref, b_ref, out_ref):
    out_ref[:, 0:_D] = a_ref[...].T
    out_ref[:, _D:2 * _D] = b_ref[...].T


def _detranspose(table):
    tt = table.T
    paired = pl.pallas_call(
        _detrans_body,
        grid=(_TC_NBLK,),
        in_specs=[
            pl.BlockSpec((_D, _TC_C), lambda i: (0, i)),
            pl.BlockSpec((_D, _TC_C),
                         lambda i: (0, jnp.minimum(i + _TC_NBLK, _TC_LASTBLK))),
        ],
        out_specs=pl.BlockSpec((_TC_C, 2 * _D), lambda i: (i, 0)),
        out_shape=jax.ShapeDtypeStruct((_SPLIT, 2 * _D), jnp.float32),
    )(tt, tt)
    return paired.reshape(2 * _SPLIT, _D)


def _fm_body(uft, ift, ue, ie, w, out,
             stage_u, stage_i, flat_u, flat_i,
             u_rows, i_rows, w_v, part, scores, sem_a, sem_b, sem_o):
    wid = lax.axis_index("s") * _NC + lax.axis_index("c")
    base = wid * _ROWS_PER_W

    pltpu.sync_copy(w, w_v)
    wb = [w_v[pl.ds(k * _L, _L)] for k in range(_ND)]
    lane = lax.iota(jnp.int32, _L)
    lo_rows = lane
    hi_rows = jnp.minimum(lane + _L, _F - 1)

    for blk in range(_NBLK):
        col0 = base + blk * _CBLK
        pltpu.sync_copy(uft.at[:, pl.ds(col0, _CBLK)], stage_u)
        pltpu.sync_copy(ift.at[:, pl.ds(col0, _CBLK)], stage_i)

        def remap(r):
            return jnp.where(r < _SPLIT, r * 2, r * 2 - (2 * _SPLIT - 1))

        def reorder_body(b2, carry, blk=blk):
            col = jnp.full((_L,), b2, jnp.int32)
            b = blk * _CBLK + b2
            off = b * _F
            flat_u[pl.ds(off, _L)] = remap(
                plsc.load_gather(stage_u, [lo_rows, col]))
            flat_u[pl.ds(off + _L, _L)] = remap(
                plsc.load_gather(stage_u, [hi_rows, col]))
            flat_i[pl.ds(off, _L)] = remap(
                plsc.load_gather(stage_i, [lo_rows, col]))
            flat_i[pl.ds(off + _L, _L)] = remap(
                plsc.load_gather(stage_i, [hi_rows, col]))
            return carry

        lax.fori_loop(0, _CBLK, reorder_body, 0)

    sems = (sem_a, sem_b)

    def issue(c, p):
        off = c * _GROWS
        pltpu.async_copy(ue.at[flat_u.at[pl.ds(off, _GROWS)]], u_rows.at[p],
                         sems[p])
        pltpu.async_copy(ie.at[flat_i.at[pl.ds(off, _GROWS)]], i_rows.at[p],
                         sems[p])

    def drain(p):
        pltpu.make_async_copy(ue.at[pl.ds(0, _GROWS)], u_rows.at[p],
                              sems[p]).wait()
        pltpu.make_async_copy(ie.at[pl.ds(0, _GROWS)], i_rows.at[p],
                              sems[p]).wait()

    issue(0, 0)

    def super_body(s, carry):
        for par in range(2):
            c = s * 2 + par

            @pl.when(c < _NCHUNK - 1)
            def _():
                issue(c + 1, 1 - par)

            drain(par)

            def row_body(b, carry2, par=par):
                rb = b * _F
                ua = [jnp.zeros((_L,), jnp.float32) for _ in range(_ND)]
                sa = [jnp.zeros((_L,), jnp.float32) for _ in range(_ND)]
                qa = [jnp.zeros((_L,), jnp.float32) for _ in range(_ND)]
                for f in range(_F):
                    for k in range(_ND):
                        ua[k] = ua[k] + u_rows[par, rb + f, pl.ds(k * _L, _L)]
                for f in range(_F):
                    for k in range(_ND):
                        x = i_rows[par, rb + f, pl.ds(k * _L, _L)]
                        sa[k] = sa[k] + x
                        qa[k] = qa[k] + x * x
                tot = jnp.zeros((_L,), jnp.float32)
                for k in range(_ND):
                    tot = tot + (ua[k] + wb[k]) * sa[k] \
                        + 0.5 * (sa[k] * sa[k] - qa[k])
                col = jnp.full((_L,), par * _CHUNK + b, jnp.int32)
                plsc.store_scatter(part, [lane, col], tot)
                return carry2

            lax.fori_loop(0, _CHUNK, row_body, 0)

        svec = part[0, :]
        for l in range(1, _L):
            svec = svec + part[l, :]
        scores[pl.ds(s * _L, _L)] = svec
        return carry

    lax.fori_loop(0, _NCHUNK // 2, super_body, 0)
    pltpu.async_copy(scores, out.at[pl.ds(base, _ROWS_PER_W)], sem_o).wait()


@jax.jit
def kernel(user_features, item_features, user_emb, item_emb, linear_w):
    mesh = plsc.VectorSubcoreMesh(core_axis_name="c", subcore_axis_name="s",
                                  num_cores=_NC, num_subcores=_NS)
    fm = functools.partial(
        pl.kernel,
        out_type=jax.ShapeDtypeStruct((_B,), jnp.float32),
        mesh=mesh,
        compiler_params=pltpu.CompilerParams(needs_layout_passes=False,
                                             use_tc_tiling_on_sc=False),
        scratch_types=[
            pltpu.VMEM((_F, _CBLK), jnp.int32),
            pltpu.VMEM((_F, _CBLK), jnp.int32),
            pltpu.VMEM((_FLAT,), jnp.int32),
            pltpu.VMEM((_FLAT,), jnp.int32),
            pltpu.VMEM((2, _GROWS, _D), jnp.float32),
            pltpu.VMEM((2, _GROWS, _D), jnp.float32),
            pltpu.VMEM((_D,), jnp.float32),
            pltpu.VMEM((_L, 2 * _CHUNK), jnp.float32),
            pltpu.VMEM((_ROWS_PER_W,), jnp.float32),
            pltpu.SemaphoreType.DMA,
            pltpu.SemaphoreType.DMA,
            pltpu.SemaphoreType.DMA,
        ],
    )(_fm_body)
    return fm(user_features.T, item_features.T, _detranspose(user_emb),
              _detranspose(item_emb), linear_w.reshape(-1))


# speedup vs baseline: 1.4047x; 1.0517x over previous
import functools

import jax
import jax.numpy as jnp
from jax import lax
from jax.experimental import pallas as pl
from jax.experimental.pallas import tpu as pltpu
from jax.experimental.pallas import tpu_sc as plsc

_NC = 2
_NS = 16
_NW = _NC * _NS
_L = 16

_B = 16384
_F = 26
_D = 64
_ND = _D // _L
_ROWS_PER_W = _B // _NW
_CHUNK = 8
_NCHUNK = _ROWS_PER_W // _CHUNK
_GROWS = _CHUNK * _F
_CBLK = 128
_NBLK = _ROWS_PER_W // _CBLK
_FLAT = _ROWS_PER_W * _F + _L // 2

_V = 1000000
_TC_C = 16384
_SPLIT = 507904
_TC_NBLK = _SPLIT // _TC_C
_TC_LASTBLK = (_V - 1) // _TC_C


def _detrans_body(a_ref, b_ref, out_ref):
    out_ref[:, 0:_D] = a_ref[...].T
    out_ref[:, _D:2 * _D] = b_ref[...].T


def _detranspose(table):
    tt = table.T
    paired = pl.pallas_call(
        _detrans_body,
        grid=(_TC_NBLK,),
        in_specs=[
            pl.BlockSpec((_D, _TC_C), lambda i: (0, i)),
            pl.BlockSpec((_D, _TC_C),
                         lambda i: (0, jnp.minimum(i + _TC_NBLK, _TC_LASTBLK))),
        ],
        out_specs=pl.BlockSpec((_TC_C, 2 * _D), lambda i: (i, 0)),
        out_shape=jax.ShapeDtypeStruct((_SPLIT, 2 * _D), jnp.float32),
    )(tt, tt)
    return paired.reshape(2 * _SPLIT, _D)


def _fm_body(uft, ift, ue, ie, w, out,
             stage_u, stage_i, flat_u, flat_i,
             u_rows, i_rows, w_v, part, scores, sem_a, sem_b, sem_o):
    wid = lax.axis_index("s") * _NC + lax.axis_index("c")
    base = wid * _ROWS_PER_W

    pltpu.sync_copy(w, w_v)
    wb = [w_v[pl.ds(k * _L, _L)] for k in range(_ND)]
    lane = lax.iota(jnp.int32, _L)
    lo_rows = lane
    hi_rows = jnp.minimum(lane + _L, _F - 1)

    for blk in range(_NBLK):
        col0 = base + blk * _CBLK
        pltpu.sync_copy(uft.at[:, pl.ds(col0, _CBLK)], stage_u)
        pltpu.sync_copy(ift.at[:, pl.ds(col0, _CBLK)], stage_i)

        def remap(r):
            return jnp.where(r < _SPLIT, r * 2, r * 2 - (2 * _SPLIT - 1))

        def reorder_body(b2, carry, blk=blk):
            col = jnp.full((_L,), b2, jnp.int32)
            b = blk * _CBLK + b2
            off = b * _F
            flat_u[pl.ds(off, _L)] = remap(
                plsc.load_gather(stage_u, [lo_rows, col]))
            flat_u[pl.ds(off + _L, _L)] = remap(
                plsc.load_gather(stage_u, [hi_rows, col]))
            flat_i[pl.ds(off, _L)] = remap(
                plsc.load_gather(stage_i, [lo_rows, col]))
            flat_i[pl.ds(off + _L, _L)] = remap(
                plsc.load_gather(stage_i, [hi_rows, col]))
            return carry

        lax.fori_loop(0, _CBLK, reorder_body, 0)

    sems = (sem_a, sem_b)

    def issue(c, p):
        off = c * _GROWS
        pltpu.async_copy(ue.at[flat_u.at[pl.ds(off, _GROWS)]], u_rows.at[p],
                         sems[p])
        pltpu.async_copy(ie.at[flat_i.at[pl.ds(off, _GROWS)]], i_rows.at[p],
                         sems[p])

    def drain(p):
        pltpu.make_async_copy(ue.at[pl.ds(0, _GROWS)], u_rows.at[p],
                              sems[p]).wait()
        pltpu.make_async_copy(ie.at[pl.ds(0, _GROWS)], i_rows.at[p],
                              sems[p]).wait()

    issue(0, 0)

    def super_body(s, carry):
        for par in range(2):
            c = s * 2 + par

            @pl.when(c < _NCHUNK - 1)
            def _():
                issue(c + 1, 1 - par)

            drain(par)

            def row_body(b, carry2, par=par):
                rb = b * _F
                ua = [jnp.zeros((_L,), jnp.float32) for _ in range(_ND)]
                sa = [jnp.zeros((_L,), jnp.float32) for _ in range(_ND)]
                qa = [jnp.zeros((_L,), jnp.float32) for _ in range(_ND)]
                for f in range(_F):
                    for k in range(_ND):
                        ua[k] = ua[k] + u_rows[par, rb + f, pl.ds(k * _L, _L)]
                for f in range(_F):
                    for k in range(_ND):
                        x = i_rows[par, rb + f, pl.ds(k * _L, _L)]
                        sa[k] = sa[k] + x
                        qa[k] = qa[k] + x * x
                tot = jnp.zeros((_L,), jnp.float32)
                for k in range(_ND):
                    tot = tot + (ua[k] + wb[k]) * sa[k] \
                        + 0.5 * (sa[k] * sa[k] - qa[k])
                col = jnp.full((_L,), par * _CHUNK + b, jnp.int32)
                plsc.store_scatter(part, [lane, col], tot)
                return carry2

            lax.fori_loop(0, _CHUNK, row_body, 0)

        svec = part[0, :]
        for l in range(1, _L):
            svec = svec + part[l, :]
        scores[pl.ds(s * _L, _L)] = svec
        return carry

    lax.fori_loop(0, _NCHUNK // 2, super_body, 0)
    pltpu.async_copy(scores, out.at[pl.ds(base, _ROWS_PER_W)], sem_o).wait()


@jax.jit
def kernel(user_features, item_features, user_emb, item_emb, linear_w):
    mesh = plsc.VectorSubcoreMesh(core_axis_name="c", subcore_axis_name="s",
                                  num_cores=_NC, num_subcores=_NS)
    fm = functools.partial(
        pl.kernel,
        out_type=jax.ShapeDtypeStruct((_B,), jnp.float32),
        mesh=mesh,
        compiler_params=pltpu.CompilerParams(needs_layout_passes=False,
                                             use_tc_tiling_on_sc=False),
        scratch_types=[
            pltpu.VMEM((_F, _CBLK), jnp.int32),
            pltpu.VMEM((_F, _CBLK), jnp.int32),
            pltpu.VMEM((_FLAT,), jnp.int32),
            pltpu.VMEM((_FLAT,), jnp.int32),
            pltpu.VMEM((2, _GROWS, _D), jnp.float32),
            pltpu.VMEM((2, _GROWS, _D), jnp.float32),
            pltpu.VMEM((_D,), jnp.float32),
            pltpu.VMEM((_L, 2 * _CHUNK), jnp.float32),
            pltpu.VMEM((_ROWS_PER_W,), jnp.float32),
            pltpu.SemaphoreType.DMA,
            pltpu.SemaphoreType.DMA,
            pltpu.SemaphoreType.DMA,
        ],
    )(_fm_body)
    return fm(user_features.T, item_features.T, _detranspose(user_emb),
              _detranspose(item_emb), linear_w.reshape(-1))


# speedup vs baseline: 1.4887x; 1.0598x over previous
import functools

import jax
import jax.numpy as jnp
from jax import lax
from jax.experimental import pallas as pl
from jax.experimental.pallas import tpu as pltpu
from jax.experimental.pallas import tpu_sc as plsc

_NC = 2
_NS = 16
_NW = _NC * _NS
_L = 16

_B = 16384
_F = 26
_D = 64
_ND = _D // _L
_ROWS_PER_W = _B // _NW
_CHUNK = 8
_NCHUNK = _ROWS_PER_W // _CHUNK
_GROWS = _CHUNK * _F
_CBLK = 128
_NBLK = _ROWS_PER_W // _CBLK
_FLAT = _ROWS_PER_W * _F + _L // 2

_V = 1000000
_TC_C = 16384
_SPLIT = 507904
_TC_NBLK = _SPLIT // _TC_C
_TC_LASTBLK = (_V - 1) // _TC_C


def _detrans_body(a_ref, b_ref, out_ref):
    out_ref[:, 0:_D] = a_ref[...].T
    out_ref[:, _D:2 * _D] = b_ref[...].T


def _detranspose(table):
    tt = table.T
    paired = pl.pallas_call(
        _detrans_body,
        grid=(_TC_NBLK,),
        in_specs=[
            pl.BlockSpec((_D, _TC_C), lambda i: (0, i)),
            pl.BlockSpec((_D, _TC_C),
                         lambda i: (0, jnp.minimum(i + _TC_NBLK, _TC_LASTBLK))),
        ],
        out_specs=pl.BlockSpec((_TC_C, 2 * _D), lambda i: (i, 0)),
        out_shape=jax.ShapeDtypeStruct((_SPLIT, 2 * _D), jnp.float32),
        compiler_params=pltpu.CompilerParams(vmem_limit_bytes=100 * 2**20),
    )(tt, tt)
    return paired.reshape(2 * _SPLIT, _D)


def _remap(r):
    return jnp.where(r < _SPLIT, r * 2, r * 2 - (2 * _SPLIT - 1))


def _reorder_indices(ft, base, stage, flat, lane):
    lo_rows = lane
    hi_rows = jnp.minimum(lane + _L, _F - 1)
    for blk in range(_NBLK):
        col0 = base + blk * _CBLK
        pltpu.sync_copy(ft.at[:, pl.ds(col0, _CBLK)], stage)

        def reorder_body(b2, carry, blk=blk):
            col = jnp.full((_L,), b2, jnp.int32)
            off = (blk * _CBLK + b2) * _F
            flat[pl.ds(off, _L)] = _remap(
                plsc.load_gather(stage, [lo_rows, col]))
            flat[pl.ds(off + _L, _L)] = _remap(
                plsc.load_gather(stage, [hi_rows, col]))
            return carry

        lax.fori_loop(0, _CBLK, reorder_body, 0)


def _colsum(part):
    svec = part[0, :]
    for l in range(1, _L):
        svec = svec + part[l, :]
    return svec


def _item_body(ift, ie, w, isum_out, ft_out,
               stage, flat, rows_a, rows_b, w_v, part, isum_v, ft_v,
               sem_a, sem_b, sem_o):
    wid = lax.axis_index("s") * _NC + lax.axis_index("c")
    base = wid * _ROWS_PER_W
    pltpu.sync_copy(w, w_v)
    wb = [w_v[pl.ds(k * _L, _L)] for k in range(_ND)]
    lane = lax.iota(jnp.int32, _L)

    _reorder_indices(ift, base, stage, flat, lane)

    sems = (sem_a, sem_b)

    def issue(c, p):
        pltpu.async_copy(ie.at[flat.at[pl.ds(c * _GROWS, _GROWS)]],
                         (rows_a, rows_b)[p], sems[p])

    def drain(p):
        pltpu.make_async_copy(ie.at[pl.ds(0, _GROWS)], (rows_a, rows_b)[p],
                              sems[p]).wait()

    issue(0, 0)

    def super_body(s, carry):
        for par in range(2):
            c = s * 2 + par

            @pl.when(c < _NCHUNK - 1)
            def _():
                issue(c + 1, 1 - par)

            drain(par)

            def row_body(b, carry2, par=par):
                buf = (rows_a, rows_b)[par]
                rb = b * _F
                sa = [jnp.zeros((_L,), jnp.float32) for _ in range(_ND)]
                qa = [jnp.zeros((_L,), jnp.float32) for _ in range(_ND)]
                for f in range(_F):
                    for k in range(_ND):
                        x = buf[rb + f, pl.ds(k * _L, _L)]
                        sa[k] = sa[k] + x
                        qa[k] = qa[k] + x * x
                tot = jnp.zeros((_L,), jnp.float32)
                for k in range(_ND):
                    tot = tot + wb[k] * sa[k] \
                        + 0.5 * (sa[k] * sa[k] - qa[k])
                    isum_v[s * _L + par * _CHUNK + b,
                           pl.ds(k * _L, _L)] = sa[k]
                col = jnp.full((_L,), par * _CHUNK + b, jnp.int32)
                plsc.store_scatter(part, [lane, col], tot)
                return carry2

            lax.fori_loop(0, _CHUNK, row_body, 0)

        ft_v[pl.ds(s * _L, _L)] = _colsum(part)
        return carry

    lax.fori_loop(0, _NCHUNK // 2, super_body, 0)
    pltpu.async_copy(isum_v, isum_out.at[pl.ds(base, _ROWS_PER_W)],
                     sem_o).wait()
    pltpu.async_copy(ft_v, ft_out.at[pl.ds(base, _ROWS_PER_W)], sem_o).wait()


def _user_body(uft, ue, isum_in, ft_in, out,
               stage, flat, rows_a, rows_b, part, isum_v, ft_v, scores,
               sem_a, sem_b, sem_o):
    wid = lax.axis_index("s") * _NC + lax.axis_index("c")
    base = wid * _ROWS_PER_W
    lane = lax.iota(jnp.int32, _L)

    pltpu.sync_copy(isum_in.at[pl.ds(base, _ROWS_PER_W)], isum_v)
    pltpu.sync_copy(ft_in.at[pl.ds(base, _ROWS_PER_W)], ft_v)
    _reorder_indices(uft, base, stage, flat, lane)

    sems = (sem_a, sem_b)

    def issue(c, p):
        pltpu.async_copy(ue.at[flat.at[pl.ds(c * _GROWS, _GROWS)]],
                         (rows_a, rows_b)[p], sems[p])

    def drain(p):
        pltpu.make_async_copy(ue.at[pl.ds(0, _GROWS)], (rows_a, rows_b)[p],
                              sems[p]).wait()

    issue(0, 0)

    def super_body(s, carry):
        for par in range(2):
            c = s * 2 + par

            @pl.when(c < _NCHUNK - 1)
            def _():
                issue(c + 1, 1 - par)

            drain(par)

            def row_body(b, carry2, par=par):
                buf = (rows_a, rows_b)[par]
                rb = b * _F
                ua = [jnp.zeros((_L,), jnp.float32) for _ in range(_ND)]
                for f in range(_F):
                    for k in range(_ND):
                        ua[k] = ua[k] + buf[rb + f, pl.ds(k * _L, _L)]
                tot = jnp.zeros((_L,), jnp.float32)
                row = s * _L + par * _CHUNK + b
                for k in range(_ND):
                    tot = tot + ua[k] * isum_v[row, pl.ds(k * _L, _L)]
                col = jnp.full((_L,), par * _CHUNK + b, jnp.int32)
                plsc.store_scatter(part, [lane, col], tot)
                return carry2

            lax.fori_loop(0, _CHUNK, row_body, 0)

        scores[pl.ds(s * _L, _L)] = _colsum(part) + ft_v[pl.ds(s * _L, _L)]
        return carry

    lax.fori_loop(0, _NCHUNK // 2, super_body, 0)
    pltpu.async_copy(scores, out.at[pl.ds(base, _ROWS_PER_W)], sem_o).wait()


@jax.jit
def kernel(user_features, item_features, user_emb, item_emb, linear_w):
    mesh = plsc.VectorSubcoreMesh(core_axis_name="c", subcore_axis_name="s",
                                  num_cores=_NC, num_subcores=_NS)
    params = pltpu.CompilerParams(needs_layout_passes=False,
                                  use_tc_tiling_on_sc=False)
    item_fm = functools.partial(
        pl.kernel,
        out_type=(jax.ShapeDtypeStruct((_B, _D), jnp.float32),
                  jax.ShapeDtypeStruct((_B,), jnp.float32)),
        mesh=mesh,
        compiler_params=params,
        scratch_types=[
            pltpu.VMEM((_F, _CBLK), jnp.int32),
            pltpu.VMEM((_FLAT,), jnp.int32),
            pltpu.VMEM((_GROWS, _D), jnp.float32),
            pltpu.VMEM((_GROWS, _D), jnp.float32),
            pltpu.VMEM((_D,), jnp.float32),
            pltpu.VMEM((_L, 2 * _CHUNK), jnp.float32),
            pltpu.VMEM((_ROWS_PER_W, _D), jnp.float32),
            pltpu.VMEM((_ROWS_PER_W,), jnp.float32),
            pltpu.SemaphoreType.DMA,
            pltpu.SemaphoreType.DMA,
            pltpu.SemaphoreType.DMA,
        ],
    )(_item_body)
    user_fm = functools.partial(
        pl.kernel,
        out_type=jax.ShapeDtypeStruct((_B,), jnp.float32),
        mesh=mesh,
        compiler_params=params,
        scratch_types=[
            pltpu.VMEM((_F, _CBLK), jnp.int32),
            pltpu.VMEM((_FLAT,), jnp.int32),
            pltpu.VMEM((_GROWS, _D), jnp.float32),
            pltpu.VMEM((_GROWS, _D), jnp.float32),
            pltpu.VMEM((_L, 2 * _CHUNK), jnp.float32),
            pltpu.VMEM((_ROWS_PER_W, _D), jnp.float32),
            pltpu.VMEM((_ROWS_PER_W,), jnp.float32),
            pltpu.VMEM((_ROWS_PER_W,), jnp.float32),
            pltpu.SemaphoreType.DMA,
            pltpu.SemaphoreType.DMA,
            pltpu.SemaphoreType.DMA,
        ],
    )(_user_body)
    ie_t = _detranspose(item_emb)
    ue_t = _detranspose(user_emb)
    isum, ft = item_fm(item_features.T, ie_t, linear_w.reshape(-1))
    return user_fm(user_features.T, ue_t, isum, ft)


# speedup vs baseline: 1.5057x; 1.0114x over previous
import functools

import jax
import jax.numpy as jnp
from jax import lax
from jax.experimental import pallas as pl
from jax.experimental.pallas import tpu as pltpu
from jax.experimental.pallas import tpu_sc as plsc

_NC = 2
_NS = 16
_NW = _NC * _NS
_L = 16

_B = 16384
_F = 26
_D = 64
_ND = _D // _L
_ROWS_PER_W = _B // _NW
_CHUNK = 16
_NCHUNK = _ROWS_PER_W // _CHUNK
_GROWS = _CHUNK * _F
_CBLK = 128
_NBLK = _ROWS_PER_W // _CBLK
_FLAT = _ROWS_PER_W * _F + _L // 2

_V = 1000000
_TC_C = 16384
_SPLIT = 507904
_TC_NBLK = _SPLIT // _TC_C
_TC_LASTBLK = (_V - 1) // _TC_C


def _detrans_body(a_ref, b_ref, out_ref):
    out_ref[:, 0:_D] = a_ref[...].T
    out_ref[:, _D:2 * _D] = b_ref[...].T


def _detranspose(table):
    tt = table.T
    paired = pl.pallas_call(
        _detrans_body,
        grid=(_TC_NBLK,),
        in_specs=[
            pl.BlockSpec((_D, _TC_C), lambda i: (0, i)),
            pl.BlockSpec((_D, _TC_C),
                         lambda i: (0, jnp.minimum(i + _TC_NBLK, _TC_LASTBLK))),
        ],
        out_specs=pl.BlockSpec((_TC_C, 2 * _D), lambda i: (i, 0)),
        out_shape=jax.ShapeDtypeStruct((_SPLIT, 2 * _D), jnp.float32),
        compiler_params=pltpu.CompilerParams(vmem_limit_bytes=100 * 2**20),
    )(tt, tt)
    return paired.reshape(2 * _SPLIT, _D)


def _remap(r):
    return jnp.where(r < _SPLIT, r * 2, r * 2 - (2 * _SPLIT - 1))


def _reorder_indices(ft, base, stage, flat, lane):
    lo_rows = lane
    hi_rows = jnp.minimum(lane + _L, _F - 1)
    for blk in range(_NBLK):
        col0 = base + blk * _CBLK
        pltpu.sync_copy(ft.at[:, pl.ds(col0, _CBLK)], stage)

        def reorder_body(b2, carry, blk=blk):
            col = jnp.full((_L,), b2, jnp.int32)
            off = (blk * _CBLK + b2) * _F
            flat[pl.ds(off, _L)] = _remap(
                plsc.load_gather(stage, [lo_rows, col]))
            flat[pl.ds(off + _L, _L)] = _remap(
                plsc.load_gather(stage, [hi_rows, col]))
            return carry

        lax.fori_loop(0, _CBLK, reorder_body, 0)


def _colsum(part):
    svec = part[0, :]
    for l in range(1, _L):
        svec = svec + part[l, :]
    return svec


def _item_body(ift, ie, w, isum_out, ft_out,
               stage, flat, rows_a, rows_b, w_v, part, isum_v, ft_v,
               sem_a, sem_b, sem_o):
    wid = lax.axis_index("s") * _NC + lax.axis_index("c")
    base = wid * _ROWS_PER_W
    pltpu.sync_copy(w, w_v)
    wb = [w_v[pl.ds(k * _L, _L)] for k in range(_ND)]
    lane = lax.iota(jnp.int32, _L)

    _reorder_indices(ift, base, stage, flat, lane)

    sems = (sem_a, sem_b)

    def issue(c, p):
        pltpu.async_copy(ie.at[flat.at[pl.ds(c * _GROWS, _GROWS)]],
                         (rows_a, rows_b)[p], sems[p])

    def drain(p):
        pltpu.make_async_copy(ie.at[pl.ds(0, _GROWS)], (rows_a, rows_b)[p],
                              sems[p]).wait()

    issue(0, 0)

    def super_body(s, carry):
        for par in range(2):
            c = s * 2 + par

            @pl.when(c < _NCHUNK - 1)
            def _():
                issue(c + 1, 1 - par)

            drain(par)

            def row_body(b, carry2, par=par):
                buf = (rows_a, rows_b)[par]
                rb = b * _F
                sa = [jnp.zeros((_L,), jnp.float32) for _ in range(_ND)]
                qa = [jnp.zeros((_L,), jnp.float32) for _ in range(_ND)]
                for f in range(_F):
                    for k in range(_ND):
                        x = buf[rb + f, pl.ds(k * _L, _L)]
                        sa[k] = sa[k] + x
                        qa[k] = qa[k] + x * x
                tot = jnp.zeros((_L,), jnp.float32)
                for k in range(_ND):
                    tot = tot + wb[k] * sa[k] \
                        + 0.5 * (sa[k] * sa[k] - qa[k])
                    isum_v[(s * 2 + par) * _CHUNK + b,
                           pl.ds(k * _L, _L)] = sa[k]
                plsc.store_scatter(part, [lane, jnp.full((_L,), b, jnp.int32)],
                                   tot)
                return carry2

            lax.fori_loop(0, _CHUNK, row_body, 0)
            ft_v[pl.ds((s * 2 + par) * _CHUNK, _CHUNK)] = _colsum(part)
        return carry

    lax.fori_loop(0, _NCHUNK // 2, super_body, 0)
    pltpu.async_copy(isum_v, isum_out.at[pl.ds(base, _ROWS_PER_W)],
                     sem_o).wait()
    pltpu.async_copy(ft_v, ft_out.at[pl.ds(base, _ROWS_PER_W)], sem_o).wait()


def _user_body(uft, ue, isum_in, ft_in, out,
               stage, flat, rows_a, rows_b, part, isum_v, ft_v, scores,
               sem_a, sem_b, sem_o):
    wid = lax.axis_index("s") * _NC + lax.axis_index("c")
    base = wid * _ROWS_PER_W
    lane = lax.iota(jnp.int32, _L)

    pltpu.sync_copy(isum_in.at[pl.ds(base, _ROWS_PER_W)], isum_v)
    pltpu.sync_copy(ft_in.at[pl.ds(base, _ROWS_PER_W)], ft_v)
    _reorder_indices(uft, base, stage, flat, lane)

    sems = (sem_a, sem_b)

    def issue(c, p):
        pltpu.async_copy(ue.at[flat.at[pl.ds(c * _GROWS, _GROWS)]],
                         (rows_a, rows_b)[p], sems[p])

    def drain(p):
        pltpu.make_async_copy(ue.at[pl.ds(0, _GROWS)], (rows_a, rows_b)[p],
                              sems[p]).wait()

    issue(0, 0)

    def super_body(s, carry):
        for par in range(2):
            c = s * 2 + par

            @pl.when(c < _NCHUNK - 1)
            def _():
                issue(c + 1, 1 - par)

            drain(par)

            def row_body(b, carry2, par=par):
                buf = (rows_a, rows_b)[par]
                rb = b * _F
                ua = [jnp.zeros((_L,), jnp.float32) for _ in range(_ND)]
                for f in range(_F):
                    for k in range(_ND):
                        ua[k] = ua[k] + buf[rb + f, pl.ds(k * _L, _L)]
                tot = jnp.zeros((_L,), jnp.float32)
                row = (s * 2 + par) * _CHUNK + b
                for k in range(_ND):
                    tot = tot + ua[k] * isum_v[row, pl.ds(k * _L, _L)]
                plsc.store_scatter(part, [lane, jnp.full((_L,), b, jnp.int32)],
                                   tot)
                return carry2

            lax.fori_loop(0, _CHUNK, row_body, 0)
            off = (s * 2 + par) * _CHUNK
            scores[pl.ds(off, _CHUNK)] = _colsum(part) \
                + ft_v[pl.ds(off, _CHUNK)]
        return carry

    lax.fori_loop(0, _NCHUNK // 2, super_body, 0)
    pltpu.async_copy(scores, out.at[pl.ds(base, _ROWS_PER_W)], sem_o).wait()


@jax.jit
def kernel(user_features, item_features, user_emb, item_emb, linear_w):
    mesh = plsc.VectorSubcoreMesh(core_axis_name="c", subcore_axis_name="s",
                                  num_cores=_NC, num_subcores=_NS)
    params = pltpu.CompilerParams(needs_layout_passes=False,
                                  use_tc_tiling_on_sc=False)
    item_fm = functools.partial(
        pl.kernel,
        out_type=(jax.ShapeDtypeStruct((_B, _D), jnp.float32),
                  jax.ShapeDtypeStruct((_B,), jnp.float32)),
        mesh=mesh,
        compiler_params=params,
        scratch_types=[
            pltpu.VMEM((_F, _CBLK), jnp.int32),
            pltpu.VMEM((_FLAT,), jnp.int32),
            pltpu.VMEM((_GROWS, _D), jnp.float32),
            pltpu.VMEM((_GROWS, _D), jnp.float32),
            pltpu.VMEM((_D,), jnp.float32),
            pltpu.VMEM((_L, _CHUNK), jnp.float32),
            pltpu.VMEM((_ROWS_PER_W, _D), jnp.float32),
            pltpu.VMEM((_ROWS_PER_W,), jnp.float32),
            pltpu.SemaphoreType.DMA,
            pltpu.SemaphoreType.DMA,
            pltpu.SemaphoreType.DMA,
        ],
    )(_item_body)
    user_fm = functools.partial(
        pl.kernel,
        out_type=jax.ShapeDtypeStruct((_B,), jnp.float32),
        mesh=mesh,
        compiler_params=params,
        scratch_types=[
            pltpu.VMEM((_F, _CBLK), jnp.int32),
            pltpu.VMEM((_FLAT,), jnp.int32),
            pltpu.VMEM((_GROWS, _D), jnp.float32),
            pltpu.VMEM((_GROWS, _D), jnp.float32),
            pltpu.VMEM((_L, _CHUNK), jnp.float32),
            pltpu.VMEM((_ROWS_PER_W, _D), jnp.float32),
            pltpu.VMEM((_ROWS_PER_W,), jnp.float32),
            pltpu.VMEM((_ROWS_PER_W,), jnp.float32),
            pltpu.SemaphoreType.DMA,
            pltpu.SemaphoreType.DMA,
            pltpu.SemaphoreType.DMA,
        ],
    )(_user_body)
    ie_t = _detranspose(item_emb)
    ue_t = _detranspose(user_emb)
    isum, ft = item_fm(item_features.T, ie_t, linear_w.reshape(-1))
    return user_fm(user_features.T, ue_t, isum, ft)


# speedup vs baseline: 1.5598x; 1.0359x over previous
import functools

import jax
import jax.numpy as jnp
from jax import lax
from jax.experimental import pallas as pl
from jax.experimental.pallas import tpu as pltpu
from jax.experimental.pallas import tpu_sc as plsc

_NC = 2
_NS = 16
_NW = _NC * _NS
_L = 16

_B = 16384
_F = 26
_D = 64
_ND = _D // _L
_ROWS_PER_W = _B // _NW
_CHUNK = 16
_NCHUNK = _ROWS_PER_W // _CHUNK
_GROWS = _CHUNK * _F
_CBLK = 128
_NBLK = _ROWS_PER_W // _CBLK
_FLAT = _ROWS_PER_W * _F + _L // 2

_V = 1000000
_TC_C = 16384
_SPLIT = 507904
_TC_NBLK = _SPLIT // _TC_C
_TC_LASTBLK = (_V - 1) // _TC_C


def _detrans_body(a_ref, b_ref, out_ref):
    out_ref[:, 0:_D] = a_ref[...].T
    out_ref[:, _D:2 * _D] = b_ref[...].T


def _detranspose(table):
    tt = table.T
    paired = pl.pallas_call(
        _detrans_body,
        grid=(_TC_NBLK,),
        in_specs=[
            pl.BlockSpec((_D, _TC_C), lambda i: (0, i)),
            pl.BlockSpec((_D, _TC_C),
                         lambda i: (0, jnp.minimum(i + _TC_NBLK, _TC_LASTBLK))),
        ],
        out_specs=pl.BlockSpec((_TC_C, 2 * _D), lambda i: (i, 0)),
        out_shape=jax.ShapeDtypeStruct((_SPLIT, 2 * _D), jnp.float32),
        compiler_params=pltpu.CompilerParams(vmem_limit_bytes=100 * 2**20),
    )(tt, tt)
    return paired.reshape(2 * _SPLIT, _D)


def _remap(r):
    return jnp.where(r < _SPLIT, r * 2, r * 2 - (2 * _SPLIT - 1))


def _reorder_indices(ft, base, stage, flat, lane):
    lo_rows = lane
    hi_rows = jnp.minimum(lane + _L, _F - 1)
    for blk in range(_NBLK):
        col0 = base + blk * _CBLK
        pltpu.sync_copy(ft.at[:, pl.ds(col0, _CBLK)], stage)

        def reorder_body(b2, carry, blk=blk):
            col = jnp.full((_L,), b2, jnp.int32)
            off = (blk * _CBLK + b2) * _F
            flat[pl.ds(off, _L)] = _remap(
                plsc.load_gather(stage, [lo_rows, col]))
            flat[pl.ds(off + _L, _L)] = _remap(
                plsc.load_gather(stage, [hi_rows, col]))
            return carry

        lax.fori_loop(0, _CBLK, reorder_body, 0)


def _colsum(part):
    svec = part[0, :]
    for l in range(1, _L):
        svec = svec + part[l, :]
    return svec


def _item_body(ift, uft, ie, w, isum_out, ft_out, uflat_out,
               stage, flat, rows_a, rows_b, w_v, part, isum_v, ft_v,
               sem_a, sem_b, sem_o):
    wid = lax.axis_index("s") * _NC + lax.axis_index("c")
    base = wid * _ROWS_PER_W
    pltpu.sync_copy(w, w_v)
    wb = [w_v[pl.ds(k * _L, _L)] for k in range(_ND)]
    lane = lax.iota(jnp.int32, _L)

    _reorder_indices(uft, base, stage, flat, lane)
    pltpu.async_copy(flat, uflat_out.at[wid], sem_o).wait()
    _reorder_indices(ift, base, stage, flat, lane)

    sems = (sem_a, sem_b)

    def issue(c, p):
        pltpu.async_copy(ie.at[flat.at[pl.ds(c * _GROWS, _GROWS)]],
                         (rows_a, rows_b)[p], sems[p])

    def drain(p):
        pltpu.make_async_copy(ie.at[pl.ds(0, _GROWS)], (rows_a, rows_b)[p],
                              sems[p]).wait()

    issue(0, 0)

    def super_body(s, carry):
        for par in range(2):
            c = s * 2 + par

            @pl.when(c < _NCHUNK - 1)
            def _():
                issue(c + 1, 1 - par)

            drain(par)

            def row_body(b, carry2, par=par):
                buf = (rows_a, rows_b)[par]
                rb = b * _F
                sa = [jnp.zeros((_L,), jnp.float32) for _ in range(_ND)]
                qa = [jnp.zeros((_L,), jnp.float32) for _ in range(_ND)]
                for f in range(_F):
                    for k in range(_ND):
                        x = buf[rb + f, pl.ds(k * _L, _L)]
                        sa[k] = sa[k] + x
                        qa[k] = qa[k] + x * x
                tot = jnp.zeros((_L,), jnp.float32)
                for k in range(_ND):
                    tot = tot + wb[k] * sa[k] \
                        + 0.5 * (sa[k] * sa[k] - qa[k])
                    isum_v[(s * 2 + par) * _CHUNK + b,
                           pl.ds(k * _L, _L)] = sa[k]
                plsc.store_scatter(part, [lane, jnp.full((_L,), b, jnp.int32)],
                                   tot)
                return carry2

            lax.fori_loop(0, _CHUNK, row_body, 0)
            ft_v[pl.ds((s * 2 + par) * _CHUNK, _CHUNK)] = _colsum(part)
        return carry

    lax.fori_loop(0, _NCHUNK // 2, super_body, 0)
    pltpu.async_copy(isum_v, isum_out.at[pl.ds(base, _ROWS_PER_W)],
                     sem_o).wait()
    pltpu.async_copy(ft_v, ft_out.at[pl.ds(base, _ROWS_PER_W)], sem_o).wait()


def _user_body(ue, isum_in, ft_in, uflat_in, out,
               flat, rows_a, rows_b, part, isum_v, ft_v, scores,
               sem_a, sem_b, sem_o):
    wid = lax.axis_index("s") * _NC + lax.axis_index("c")
    base = wid * _ROWS_PER_W
    lane = lax.iota(jnp.int32, _L)

    pltpu.sync_copy(uflat_in.at[wid], flat)

    sems = (sem_a, sem_b)

    def issue(c, p):
        pltpu.async_copy(ue.at[flat.at[pl.ds(c * _GROWS, _GROWS)]],
                         (rows_a, rows_b)[p], sems[p])

    def drain(p):
        pltpu.make_async_copy(ue.at[pl.ds(0, _GROWS)], (rows_a, rows_b)[p],
                              sems[p]).wait()

    issue(0, 0)
    pltpu.sync_copy(isum_in.at[pl.ds(base, _ROWS_PER_W)], isum_v)
    pltpu.sync_copy(ft_in.at[pl.ds(base, _ROWS_PER_W)], ft_v)

    def super_body(s, carry):
        for par in range(2):
            c = s * 2 + par

            @pl.when(c < _NCHUNK - 1)
            def _():
                issue(c + 1, 1 - par)

            drain(par)

            def row_body(b, carry2, par=par):
                buf = (rows_a, rows_b)[par]
                rb = b * _F
                ua = [jnp.zeros((_L,), jnp.float32) for _ in range(_ND)]
                for f in range(_F):
                    for k in range(_ND):
                        ua[k] = ua[k] + buf[rb + f, pl.ds(k * _L, _L)]
                tot = jnp.zeros((_L,), jnp.float32)
                row = (s * 2 + par) * _CHUNK + b
                for k in range(_ND):
                    tot = tot + ua[k] * isum_v[row, pl.ds(k * _L, _L)]
                plsc.store_scatter(part, [lane, jnp.full((_L,), b, jnp.int32)],
                                   tot)
                return carry2

            lax.fori_loop(0, _CHUNK, row_body, 0)
            off = (s * 2 + par) * _CHUNK
            scores[pl.ds(off, _CHUNK)] = _colsum(part) \
                + ft_v[pl.ds(off, _CHUNK)]
        return carry

    lax.fori_loop(0, _NCHUNK // 2, super_body, 0)
    pltpu.async_copy(scores, out.at[pl.ds(base, _ROWS_PER_W)], sem_o).wait()


@jax.jit
def kernel(user_features, item_features, user_emb, item_emb, linear_w):
    mesh = plsc.VectorSubcoreMesh(core_axis_name="c", subcore_axis_name="s",
                                  num_cores=_NC, num_subcores=_NS)
    params = pltpu.CompilerParams(needs_layout_passes=False,
                                  use_tc_tiling_on_sc=False)
    item_fm = functools.partial(
        pl.kernel,
        out_type=(jax.ShapeDtypeStruct((_B, _D), jnp.float32),
                  jax.ShapeDtypeStruct((_B,), jnp.float32),
                  jax.ShapeDtypeStruct((_NW, _FLAT), jnp.int32)),
        mesh=mesh,
        compiler_params=params,
        scratch_types=[
            pltpu.VMEM((_F, _CBLK), jnp.int32),
            pltpu.VMEM((_FLAT,), jnp.int32),
            pltpu.VMEM((_GROWS, _D), jnp.float32),
            pltpu.VMEM((_GROWS, _D), jnp.float32),
            pltpu.VMEM((_D,), jnp.float32),
            pltpu.VMEM((_L, _CHUNK), jnp.float32),
            pltpu.VMEM((_ROWS_PER_W, _D), jnp.float32),
            pltpu.VMEM((_ROWS_PER_W,), jnp.float32),
            pltpu.SemaphoreType.DMA,
            pltpu.SemaphoreType.DMA,
            pltpu.SemaphoreType.DMA,
        ],
    )(_item_body)
    user_fm = functools.partial(
        pl.kernel,
        out_type=jax.ShapeDtypeStruct((_B,), jnp.float32),
        mesh=mesh,
        compiler_params=params,
        scratch_types=[
            pltpu.VMEM((_FLAT,), jnp.int32),
            pltpu.VMEM((_GROWS, _D), jnp.float32),
            pltpu.VMEM((_GROWS, _D), jnp.float32),
            pltpu.VMEM((_L, _CHUNK), jnp.float32),
            pltpu.VMEM((_ROWS_PER_W, _D), jnp.float32),
            pltpu.VMEM((_ROWS_PER_W,), jnp.float32),
            pltpu.VMEM((_ROWS_PER_W,), jnp.float32),
            pltpu.SemaphoreType.DMA,
            pltpu.SemaphoreType.DMA,
            pltpu.SemaphoreType.DMA,
        ],
    )(_user_body)
    ie_t = _detranspose(item_emb)
    ue_t = _detranspose(user_emb)
    isum, ft, uflat = item_fm(item_features.T, user_features.T, ie_t,
                              linear_w.reshape(-1))
    return user_fm(ue_t, isum, ft, uflat)
